# baseline (speedup 1.0000x reference)
import functools

import jax
import jax.numpy as jnp
from jax import lax
from jax.experimental import pallas as pl
from jax.experimental.pallas import tpu as pltpu
from jax.experimental.pallas import tpu_sc as plsc

NC = 2
NS = 16
NW = NC * NS
EB = 256
LW = 128
BLK = 1024


def _ceil_to(v, m):
    return (v + m - 1) // m * m


@functools.lru_cache(maxsize=None)
def _deg_sc(np_, k):
    rt = np_ // NS
    mesh = plsc.VectorSubcoreMesh(core_axis_name="c", subcore_axis_name="s")

    @functools.partial(
        pl.kernel,
        out_type=jax.ShapeDtypeStruct((NC * np_, LW), jnp.float32),
        mesh=mesh,
        scratch_types=[
            pltpu.VMEM((EB,), jnp.int32),
            pltpu.VMEM((EB, LW), jnp.float32),
            pltpu.VMEM_SHARED((np_, LW), jnp.float32),
        ],
    )
    def deg_kernel(dst_hbm, ones_hbm, zeros_hbm, out_hbm, dst_v, ones_v, acc_sh):
        c = lax.axis_index("c")
        s = lax.axis_index("s")
        wid = c * NS + s
        r0 = s * rt
        pltpu.sync_copy(zeros_hbm.at[pl.ds(r0, rt)], acc_sh.at[pl.ds(r0, rt)])
        pltpu.sync_copy(ones_hbm, ones_v)
        plsc.subcore_barrier()

        def step(j, carry):
            pltpu.sync_copy(dst_hbm.at[wid * k + j], dst_v)
            pltpu.sync_copy(ones_v, acc_sh.at[dst_v], add=True)
            return carry

        lax.fori_loop(0, k, step, 0)
        plsc.subcore_barrier()
        pltpu.sync_copy(acc_sh.at[pl.ds(r0, rt)], out_hbm.at[pl.ds(c * np_ + r0, rt)])

    return deg_kernel


@functools.lru_cache(maxsize=None)
def _agg_sc(np_, k):
    rt = np_ // NS
    mesh = plsc.VectorSubcoreMesh(core_axis_name="c", subcore_axis_name="s")

    @functools.partial(
        pl.kernel,
        out_type=jax.ShapeDtypeStruct((NC * np_, LW), jnp.float32),
        mesh=mesh,
        scratch_types=[
            pltpu.VMEM((EB,), jnp.int32),
            pltpu.VMEM((EB,), jnp.int32),
            pltpu.VMEM((EB, LW), jnp.float32),
            pltpu.VMEM_SHARED((np_, LW), jnp.float32),
            pltpu.SemaphoreType.DMA,
        ],
    )
    def agg_kernel(src_hbm, dst_hbm, y_hbm, zeros_hbm, out_hbm,
                   src_v, dst_v, rows_v, acc_sh, sem):
        c = lax.axis_index("c")
        s = lax.axis_index("s")
        wid = c * NS + s
        r0 = s * rt
        pltpu.sync_copy(zeros_hbm.at[pl.ds(r0, rt)], acc_sh.at[pl.ds(r0, rt)])
        plsc.subcore_barrier()

        def step(j, carry):
            row = wid * k + j
            pltpu.sync_copy(src_hbm.at[row], src_v)
            pltpu.sync_copy(dst_hbm.at[row], dst_v)
            pltpu.async_copy(y_hbm.at[src_v], rows_v, sem).wait()
            pltpu.sync_copy(rows_v, acc_sh.at[dst_v], add=True)
            return carry

        lax.fori_loop(0, k, step, 0)
        plsc.subcore_barrier()
        pltpu.sync_copy(acc_sh.at[pl.ds(r0, rt)], out_hbm.at[pl.ds(c * np_ + r0, rt)])

    return agg_kernel


def _deg_call(dst, ones, zeros):
    np_ = zeros.shape[0]
    k = dst.shape[0] // NW
    return _deg_sc(np_, k)(dst, ones, zeros)


def _agg_call(src, dst, y, zeros):
    np_ = zeros.shape[0]
    k = src.shape[0] // NW
    return _agg_sc(np_, k)(src, dst, y, zeros)


def _row_specs(n, shapes):
    return [pl.BlockSpec((BLK, w), lambda i: (i, 0)) for w in shapes]


def _pre_tc(xp, w_in, b_in, wc0, deg0, deg1):
    np_, f_in = xp.shape

    def body(x_ref, win_ref, bin_ref, wc_ref, d0_ref, d1_ref,
             h_ref, y_ref, dis_ref):
        h0 = jnp.dot(x_ref[...], win_ref[...],
                     preferred_element_type=jnp.float32) + bin_ref[...]
        dis = lax.rsqrt(d0_ref[...] + d1_ref[...] + 1.0)
        h_ref[...] = h0
        dis_ref[...] = dis
        y_ref[...] = jnp.dot(h0, wc_ref[...],
                             preferred_element_type=jnp.float32) * dis

    return pl.pallas_call(
        body,
        grid=(np_ // BLK,),
        in_specs=[
            pl.BlockSpec((BLK, f_in), lambda i: (i, 0)),
            pl.BlockSpec((f_in, LW), lambda i: (0, 0)),
            pl.BlockSpec((1, LW), lambda i: (0, 0)),
            pl.BlockSpec((LW, LW), lambda i: (0, 0)),
            pl.BlockSpec((BLK, LW), lambda i: (i, 0)),
            pl.BlockSpec((BLK, LW), lambda i: (i, 0)),
        ],
        out_specs=[
            pl.BlockSpec((BLK, LW), lambda i: (i, 0)),
            pl.BlockSpec((BLK, LW), lambda i: (i, 0)),
            pl.BlockSpec((BLK, LW), lambda i: (i, 0)),
        ],
        out_shape=[
            jax.ShapeDtypeStruct((np_, LW), jnp.float32),
            jax.ShapeDtypeStruct((np_, LW), jnp.float32),
            jax.ShapeDtypeStruct((np_, LW), jnp.float32),
        ],
    )(xp, w_in, b_in, wc0, deg0, deg1)


def _mid_tc(hs, y, agg0, agg1, dis, b_cur, w_next):
    np_ = hs.shape[0]

    def body(h_ref, y_ref, a0_ref, a1_ref, dis_ref, b_ref, w_ref,
             ho_ref, yo_ref):
        upd = jax.nn.relu(dis_ref[...] * (a0_ref[...] + a1_ref[...] + y_ref[...])
                          + b_ref[...])
        hn = h_ref[...] + upd
        ho_ref[...] = hn
        yo_ref[...] = jnp.dot(hn, w_ref[...],
                              preferred_element_type=jnp.float32) * dis_ref[...]

    return pl.pallas_call(
        body,
        grid=(np_ // BLK,),
        in_specs=[
            pl.BlockSpec((BLK, LW), lambda i: (i, 0)),
            pl.BlockSpec((BLK, LW), lambda i: (i, 0)),
            pl.BlockSpec((BLK, LW), lambda i: (i, 0)),
            pl.BlockSpec((BLK, LW), lambda i: (i, 0)),
            pl.BlockSpec((BLK, LW), lambda i: (i, 0)),
            pl.BlockSpec((1, LW), lambda i: (0, 0)),
            pl.BlockSpec((LW, LW), lambda i: (0, 0)),
        ],
        out_specs=[
            pl.BlockSpec((BLK, LW), lambda i: (i, 0)),
            pl.BlockSpec((BLK, LW), lambda i: (i, 0)),
        ],
        out_shape=[
            jax.ShapeDtypeStruct((np_, LW), jnp.float32),
            jax.ShapeDtypeStruct((np_, LW), jnp.float32),
        ],
    )(hs, y, agg0, agg1, dis, b_cur, w_next)


def _last_tc(hs, y, agg0, agg1, dis, b_cur, w_out, b_out):
    np_ = hs.shape[0]
    c_out = w_out.shape[1]

    def body(h_ref, y_ref, a0_ref, a1_ref, dis_ref, b_ref, w_ref, bo_ref,
             o_ref):
        upd = jax.nn.relu(dis_ref[...] * (a0_ref[...] + a1_ref[...] + y_ref[...])
                          + b_ref[...])
        hn = h_ref[...] + upd
        o_ref[...] = jnp.dot(hn, w_ref[...],
                             preferred_element_type=jnp.float32) + bo_ref[...]

    return pl.pallas_call(
        body,
        grid=(np_ // BLK,),
        in_specs=[
            pl.BlockSpec((BLK, LW), lambda i: (i, 0)),
            pl.BlockSpec((BLK, LW), lambda i: (i, 0)),
            pl.BlockSpec((BLK, LW), lambda i: (i, 0)),
            pl.BlockSpec((BLK, LW), lambda i: (i, 0)),
            pl.BlockSpec((BLK, LW), lambda i: (i, 0)),
            pl.BlockSpec((1, LW), lambda i: (0, 0)),
            pl.BlockSpec((LW, c_out), lambda i: (0, 0)),
            pl.BlockSpec((1, c_out), lambda i: (0, 0)),
        ],
        out_specs=[pl.BlockSpec((BLK, c_out), lambda i: (i, 0))],
        out_shape=[jax.ShapeDtypeStruct((np_, c_out), jnp.float32)],
    )(hs, y, agg0, agg1, dis, b_cur, w_out, b_out)


def _pad2(a, rows, cols):
    out = jnp.zeros((rows, cols), a.dtype)
    return out.at[:a.shape[0], :a.shape[1]].set(a)


def kernel(x, edge_index, W_in, b_in, Wc0, bc0, Wc1, bc1, Wc2, bc2, Wc3, bc3,
           W_out, b_out):
    n, f_in = x.shape
    e = edge_index.shape[1]
    np_ = _ceil_to(n + 1, BLK)
    k = -(-e // (NW * EB))
    ep = k * NW * EB

    pad_idx = jnp.full((ep - e,), n, dtype=edge_index.dtype)
    src = jnp.concatenate([edge_index[0], pad_idx]).reshape(NW * k, EB)
    dst = jnp.concatenate([edge_index[1], pad_idx]).reshape(NW * k, EB)
    xp = _pad2(x, np_, f_in)
    ones = jnp.ones((EB, LW), jnp.float32)
    zeros = jnp.zeros((np_, LW), jnp.float32)

    w_in_p = _pad2(W_in, f_in, LW)
    b_in_p = _pad2(b_in.reshape(1, -1), 1, LW)
    wcs = [_pad2(w, LW, LW) for w in (Wc0, Wc1, Wc2, Wc3)]
    bcs = [_pad2(b.reshape(1, -1), 1, LW) for b in (bc0, bc1, bc2, bc3)]
    w_out_p = _pad2(W_out, LW, W_out.shape[1])
    b_out_p = b_out.reshape(1, -1)

    deg = _deg_call(dst, ones, zeros)
    hs, y, dis = _pre_tc(xp, w_in_p, b_in_p, wcs[0], deg[:np_], deg[np_:])
    for li in range(3):
        agg = _agg_call(src, dst, y, zeros)
        hs, y = _mid_tc(hs, y, agg[:np_], agg[np_:], dis, bcs[li], wcs[li + 1])
    agg = _agg_call(src, dst, y, zeros)
    out, = _last_tc(hs, y, agg[:np_], agg[np_:], dis, bcs[3], w_out_p, b_out_p)
    return out[:n]


# speedup vs baseline: 1.3356x; 1.3356x over previous
import functools

import jax
import jax.numpy as jnp
from jax import lax
from jax.experimental import pallas as pl
from jax.experimental.pallas import tpu as pltpu
from jax.experimental.pallas import tpu_sc as plsc

NC = 2
NS = 16
NW = NC * NS
EB = 128
LW = 128
BLK = 1024


def _ceil_to(v, m):
    return (v + m - 1) // m * m


@functools.lru_cache(maxsize=None)
def _deg_sc(np_, k):
    rt = np_ // NS
    mesh = plsc.VectorSubcoreMesh(core_axis_name="c", subcore_axis_name="s")

    @functools.partial(
        pl.kernel,
        out_type=jax.ShapeDtypeStruct((NC * np_, LW), jnp.float32),
        mesh=mesh,
        scratch_types=[
            pltpu.VMEM((EB,), jnp.int32),
            pltpu.VMEM((EB, LW), jnp.float32),
            pltpu.VMEM_SHARED((np_, LW), jnp.float32),
        ],
    )
    def deg_kernel(dst_hbm, ones_hbm, zeros_hbm, out_hbm, dst_v, ones_v, acc_sh):
        c = lax.axis_index("c")
        s = lax.axis_index("s")
        wid = c * NS + s
        r0 = s * rt
        pltpu.sync_copy(zeros_hbm.at[pl.ds(r0, rt)], acc_sh.at[pl.ds(r0, rt)])
        pltpu.sync_copy(ones_hbm, ones_v)
        plsc.subcore_barrier()

        def step(j, carry):
            pltpu.sync_copy(dst_hbm.at[wid * k + j], dst_v)
            pltpu.sync_copy(ones_v, acc_sh.at[dst_v], add=True)
            return carry

        lax.fori_loop(0, k, step, 0)
        plsc.subcore_barrier()
        pltpu.sync_copy(acc_sh.at[pl.ds(r0, rt)], out_hbm.at[pl.ds(c * np_ + r0, rt)])

    return deg_kernel


@functools.lru_cache(maxsize=None)
def _agg_sc(np_, k):
    rt = np_ // NS
    mesh = plsc.VectorSubcoreMesh(core_axis_name="c", subcore_axis_name="s")

    @functools.partial(
        pl.kernel,
        out_type=jax.ShapeDtypeStruct((NC * np_, LW), jnp.float32),
        mesh=mesh,
        scratch_types=[
            pltpu.VMEM((EB,), jnp.int32),
            pltpu.VMEM((EB,), jnp.int32),
            pltpu.VMEM((EB,), jnp.int32),
            pltpu.VMEM((EB,), jnp.int32),
            pltpu.VMEM((EB, LW), jnp.float32),
            pltpu.VMEM((EB, LW), jnp.float32),
            pltpu.VMEM_SHARED((np_, LW), jnp.float32),
            pltpu.SemaphoreType.DMA,
            pltpu.SemaphoreType.DMA,
            pltpu.SemaphoreType.DMA,
            pltpu.SemaphoreType.DMA,
        ],
    )
    def agg_kernel(src_hbm, dst_hbm, y_hbm, zeros_hbm, out_hbm,
                   src_v0, dst_v0, src_v1, dst_v1, rows_v0, rows_v1, acc_sh,
                   sem_g0, sem_g1, sem_s0, sem_s1):
        c = lax.axis_index("c")
        s = lax.axis_index("s")
        wid = c * NS + s
        r0 = s * rt
        bufs = ((src_v0, dst_v0, rows_v0, sem_g0, sem_s0),
                (src_v1, dst_v1, rows_v1, sem_g1, sem_s1))
        pltpu.sync_copy(zeros_hbm.at[pl.ds(r0, rt)], acc_sh.at[pl.ds(r0, rt)])
        plsc.subcore_barrier()

        for b, (sv, dv, rv, sg, ss) in enumerate(bufs):
            pltpu.sync_copy(src_hbm.at[wid * k + b], sv)
            pltpu.sync_copy(dst_hbm.at[wid * k + b], dv)
            pltpu.async_copy(y_hbm.at[sv], rv, sg)

        def step(i, carry):
            for b, (sv, dv, rv, sg, ss) in enumerate(bufs):
                j = 2 * i + b
                pltpu.make_async_copy(y_hbm.at[sv], rv, sg).wait()
                pltpu.async_copy(rv, acc_sh.at[dv], ss, add=True)

                @pl.when(j + 2 < k)
                def _():
                    pltpu.make_async_copy(rv, acc_sh.at[dv], ss).wait()
                    pltpu.sync_copy(src_hbm.at[wid * k + j + 2], sv)
                    pltpu.sync_copy(dst_hbm.at[wid * k + j + 2], dv)
                    pltpu.async_copy(y_hbm.at[sv], rv, sg)

            return carry

        lax.fori_loop(0, k // 2, step, 0)
        for b, (sv, dv, rv, sg, ss) in enumerate(bufs):
            pltpu.make_async_copy(rv, acc_sh.at[dv], ss).wait()
        plsc.subcore_barrier()
        pltpu.sync_copy(acc_sh.at[pl.ds(r0, rt)], out_hbm.at[pl.ds(c * np_ + r0, rt)])

    return agg_kernel


def _deg_call(dst, ones, zeros):
    np_ = zeros.shape[0]
    k = dst.shape[0] // NW
    return _deg_sc(np_, k)(dst, ones, zeros)


def _agg_call(src, dst, y, zeros):
    np_ = zeros.shape[0]
    k = src.shape[0] // NW
    return _agg_sc(np_, k)(src, dst, y, zeros)


def _row_specs(n, shapes):
    return [pl.BlockSpec((BLK, w), lambda i: (i, 0)) for w in shapes]


def _pre_tc(xp, w_in, b_in, wc0, deg0, deg1):
    np_, f_in = xp.shape

    def body(x_ref, win_ref, bin_ref, wc_ref, d0_ref, d1_ref,
             h_ref, y_ref, dis_ref):
        h0 = jnp.dot(x_ref[...], win_ref[...],
                     preferred_element_type=jnp.float32) + bin_ref[...]
        dis = lax.rsqrt(d0_ref[...] + d1_ref[...] + 1.0)
        h_ref[...] = h0
        dis_ref[...] = dis
        y_ref[...] = jnp.dot(h0, wc_ref[...],
                             preferred_element_type=jnp.float32) * dis

    return pl.pallas_call(
        body,
        grid=(np_ // BLK,),
        in_specs=[
            pl.BlockSpec((BLK, f_in), lambda i: (i, 0)),
            pl.BlockSpec((f_in, LW), lambda i: (0, 0)),
            pl.BlockSpec((1, LW), lambda i: (0, 0)),
            pl.BlockSpec((LW, LW), lambda i: (0, 0)),
            pl.BlockSpec((BLK, LW), lambda i: (i, 0)),
            pl.BlockSpec((BLK, LW), lambda i: (i, 0)),
        ],
        out_specs=[
            pl.BlockSpec((BLK, LW), lambda i: (i, 0)),
            pl.BlockSpec((BLK, LW), lambda i: (i, 0)),
            pl.BlockSpec((BLK, LW), lambda i: (i, 0)),
        ],
        out_shape=[
            jax.ShapeDtypeStruct((np_, LW), jnp.float32),
            jax.ShapeDtypeStruct((np_, LW), jnp.float32),
            jax.ShapeDtypeStruct((np_, LW), jnp.float32),
        ],
    )(xp, w_in, b_in, wc0, deg0, deg1)


def _mid_tc(hs, y, agg0, agg1, dis, b_cur, w_next):
    np_ = hs.shape[0]

    def body(h_ref, y_ref, a0_ref, a1_ref, dis_ref, b_ref, w_ref,
             ho_ref, yo_ref):
        upd = jax.nn.relu(dis_ref[...] * (a0_ref[...] + a1_ref[...] + y_ref[...])
                          + b_ref[...])
        hn = h_ref[...] + upd
        ho_ref[...] = hn
        yo_ref[...] = jnp.dot(hn, w_ref[...],
                              preferred_element_type=jnp.float32) * dis_ref[...]

    return pl.pallas_call(
        body,
        grid=(np_ // BLK,),
        in_specs=[
            pl.BlockSpec((BLK, LW), lambda i: (i, 0)),
            pl.BlockSpec((BLK, LW), lambda i: (i, 0)),
            pl.BlockSpec((BLK, LW), lambda i: (i, 0)),
            pl.BlockSpec((BLK, LW), lambda i: (i, 0)),
            pl.BlockSpec((BLK, LW), lambda i: (i, 0)),
            pl.BlockSpec((1, LW), lambda i: (0, 0)),
            pl.BlockSpec((LW, LW), lambda i: (0, 0)),
        ],
        out_specs=[
            pl.BlockSpec((BLK, LW), lambda i: (i, 0)),
            pl.BlockSpec((BLK, LW), lambda i: (i, 0)),
        ],
        out_shape=[
            jax.ShapeDtypeStruct((np_, LW), jnp.float32),
            jax.ShapeDtypeStruct((np_, LW), jnp.float32),
        ],
    )(hs, y, agg0, agg1, dis, b_cur, w_next)


def _last_tc(hs, y, agg0, agg1, dis, b_cur, w_out, b_out):
    np_ = hs.shape[0]
    c_out = w_out.shape[1]

    def body(h_ref, y_ref, a0_ref, a1_ref, dis_ref, b_ref, w_ref, bo_ref,
             o_ref):
        upd = jax.nn.relu(dis_ref[...] * (a0_ref[...] + a1_ref[...] + y_ref[...])
                          + b_ref[...])
        hn = h_ref[...] + upd
        o_ref[...] = jnp.dot(hn, w_ref[...],
                             preferred_element_type=jnp.float32) + bo_ref[...]

    return pl.pallas_call(
        body,
        grid=(np_ // BLK,),
        in_specs=[
            pl.BlockSpec((BLK, LW), lambda i: (i, 0)),
            pl.BlockSpec((BLK, LW), lambda i: (i, 0)),
            pl.BlockSpec((BLK, LW), lambda i: (i, 0)),
            pl.BlockSpec((BLK, LW), lambda i: (i, 0)),
            pl.BlockSpec((BLK, LW), lambda i: (i, 0)),
            pl.BlockSpec((1, LW), lambda i: (0, 0)),
            pl.BlockSpec((LW, c_out), lambda i: (0, 0)),
            pl.BlockSpec((1, c_out), lambda i: (0, 0)),
        ],
        out_specs=[pl.BlockSpec((BLK, c_out), lambda i: (i, 0))],
        out_shape=[jax.ShapeDtypeStruct((np_, c_out), jnp.float32)],
    )(hs, y, agg0, agg1, dis, b_cur, w_out, b_out)


def _pad2(a, rows, cols):
    out = jnp.zeros((rows, cols), a.dtype)
    return out.at[:a.shape[0], :a.shape[1]].set(a)


def kernel(x, edge_index, W_in, b_in, Wc0, bc0, Wc1, bc1, Wc2, bc2, Wc3, bc3,
           W_out, b_out):
    n, f_in = x.shape
    e = edge_index.shape[1]
    np_ = _ceil_to(n + 1, BLK)
    k = -(-e // (NW * EB))
    k += k % 2
    ep = k * NW * EB

    pad_idx = jnp.full((ep - e,), n, dtype=edge_index.dtype)
    src = jnp.concatenate([edge_index[0], pad_idx]).reshape(NW * k, EB)
    dst = jnp.concatenate([edge_index[1], pad_idx]).reshape(NW * k, EB)
    xp = _pad2(x, np_, f_in)
    ones = jnp.ones((EB, LW), jnp.float32)
    zeros = jnp.zeros((np_, LW), jnp.float32)

    w_in_p = _pad2(W_in, f_in, LW)
    b_in_p = _pad2(b_in.reshape(1, -1), 1, LW)
    wcs = [_pad2(w, LW, LW) for w in (Wc0, Wc1, Wc2, Wc3)]
    bcs = [_pad2(b.reshape(1, -1), 1, LW) for b in (bc0, bc1, bc2, bc3)]
    w_out_p = _pad2(W_out, LW, W_out.shape[1])
    b_out_p = b_out.reshape(1, -1)

    deg = _deg_call(dst, ones, zeros)
    hs, y, dis = _pre_tc(xp, w_in_p, b_in_p, wcs[0], deg[:np_], deg[np_:])
    for li in range(3):
        agg = _agg_call(src, dst, y, zeros)
        hs, y = _mid_tc(hs, y, agg[:np_], agg[np_:], dis, bcs[li], wcs[li + 1])
    agg = _agg_call(src, dst, y, zeros)
    out, = _last_tc(hs, y, agg[:np_], agg[np_:], dis, bcs[3], w_out_p, b_out_p)
    return out[:n]
